# Initial kernel scaffold; baseline (speedup 1.0000x reference)
import jax
import jax.numpy as jnp
from jax.experimental import pallas as pl


def kernel(x_author, x_paper, params, edge_index_writes, edge_index_cites, edge_index_rev_writes):
    raise NotImplementedError("write your pallas kernel here")


# speedup vs baseline: 1.3268x; 1.3268x over previous
import functools
import jax
import jax.numpy as jnp
from jax.experimental import pallas as pl
from jax.experimental.pallas import tpu as pltpu

_H = 4
_D = 32
_HID = 128
_SQRT_D = 32.0 ** 0.5


def _mm_body(x_ref, w_ref, b_ref, o_ref, *, act):
    acc = jnp.dot(x_ref[...], w_ref[...], preferred_element_type=jnp.float32)
    acc = acc + b_ref[...]
    if act == "relu":
        acc = jnp.maximum(acc, 0.0)
    o_ref[...] = acc


def _matmul(x, w, b, act=None, bn=2000):
    n, din = x.shape
    f = w.shape[1]
    return pl.pallas_call(
        functools.partial(_mm_body, act=act),
        grid=(n // bn,),
        in_specs=[
            pl.BlockSpec((bn, din), lambda i: (i, 0)),
            pl.BlockSpec((din, f), lambda i: (0, 0)),
            pl.BlockSpec((1, f), lambda i: (0, 0)),
        ],
        out_specs=pl.BlockSpec((bn, f), lambda i: (i, 0)),
        out_shape=jax.ShapeDtypeStruct((n, f), jnp.float32),
    )(x, w, b.reshape(1, f))


def _out_body(num1_ref, den1_ref, num2_ref, den2_ref, h_ref, w_ref, b_ref,
              e_ref, beta_ref, o_ref):
    e = e_ref[...]
    den1 = jnp.dot(den1_ref[...], e, preferred_element_type=jnp.float32)
    x = num1_ref[...] / (den1 + 1e-16)
    if num2_ref is not None:
        den2 = jnp.dot(den2_ref[...], e, preferred_element_type=jnp.float32)
        x = x + num2_ref[...] / (den2 + 1e-16)
    g = jax.nn.gelu(x)
    o = jnp.dot(g, w_ref[...], preferred_element_type=jnp.float32) + b_ref[...]
    beta = beta_ref[...]
    o_ref[...] = beta * o + (1.0 - beta) * h_ref[...]


def _out_stage(pairs, h, w, b, beta, bn=2000):
    n = h.shape[0]
    expand = jnp.kron(jnp.eye(_H, dtype=jnp.float32), jnp.ones((1, _D), jnp.float32))
    if len(pairs) == 1:
        (num1, den1), = pairs
        body = functools.partial(_nd1_body)
        args = [num1, den1]
        nd_specs = [
            pl.BlockSpec((bn, _HID), lambda i: (i, 0)),
            pl.BlockSpec((bn, _H), lambda i: (i, 0)),
        ]
    else:
        (num1, den1), (num2, den2) = pairs
        body = _out_body
        args = [num1, den1, num2, den2]
        nd_specs = [
            pl.BlockSpec((bn, _HID), lambda i: (i, 0)),
            pl.BlockSpec((bn, _H), lambda i: (i, 0)),
            pl.BlockSpec((bn, _HID), lambda i: (i, 0)),
            pl.BlockSpec((bn, _H), lambda i: (i, 0)),
        ]
    return pl.pallas_call(
        body,
        grid=(n // bn,),
        in_specs=nd_specs + [
            pl.BlockSpec((bn, _HID), lambda i: (i, 0)),
            pl.BlockSpec((_HID, _HID), lambda i: (0, 0)),
            pl.BlockSpec((1, _HID), lambda i: (0, 0)),
            pl.BlockSpec((_H, _HID), lambda i: (0, 0)),
            pl.BlockSpec((1, 1), lambda i: (0, 0)),
        ],
        out_specs=pl.BlockSpec((bn, _HID), lambda i: (i, 0)),
        out_shape=jax.ShapeDtypeStruct((n, _HID), jnp.float32),
    )(*args, h, w, b.reshape(1, _HID), expand, beta.reshape(1, 1))


def _nd1_body(num1_ref, den1_ref, h_ref, w_ref, b_ref, e_ref, beta_ref, o_ref):
    _out_body(num1_ref, den1_ref, None, None, h_ref, w_ref, b_ref, e_ref,
              beta_ref, o_ref)


def _compose(p, rel, scale=None):
    w = p["W"].reshape(_HID, _H, _D)
    b = p["b"].reshape(_H, _D)
    wc = jnp.einsum("nhd,hde->nhe", w, rel)
    bc = jnp.einsum("hd,hde->he", b, rel)
    if scale is not None:
        wc = wc * scale[None, :, None]
        bc = bc * scale[:, None]
    return wc.reshape(_HID, _HID), bc.reshape(_HID)


def _edge_pass(q, k, v, src, dst, n_dst):
    qh = q[dst].reshape(-1, _H, _D)
    kh = k[src].reshape(-1, _H, _D)
    s = (qh * kh).sum(-1)
    w = jnp.exp(s)
    den = jax.ops.segment_sum(w, dst, num_segments=n_dst)
    msg = v[src].reshape(-1, _H, _D) * w[:, :, None]
    num = jax.ops.segment_sum(msg, dst, num_segments=n_dst)
    return num.reshape(n_dst, _HID), den


def kernel(x_author, x_paper, params, edge_index_writes, edge_index_cites, edge_index_rev_writes):
    n_a = x_author.shape[0]
    n_p = x_paper.shape[0]
    src_w, dst_w = edge_index_writes[0], edge_index_writes[1]
    src_c, dst_c = edge_index_cites[0], edge_index_cites[1]
    src_r, dst_r = edge_index_rev_writes[0], edge_index_rev_writes[1]

    lin = params["in_lin"]
    h_a = _matmul(x_author, lin["author"]["W"], lin["author"]["b"], act="relu")
    h_p = _matmul(x_paper, lin["paper"]["W"], lin["paper"]["b"], act="relu")

    for lp in params["layers"]:
        kw_w, kw_b = _compose(lp["k"]["author"], lp["a_rel"]["writes"],
                              scale=lp["p_rel"]["writes"] / _SQRT_D)
        vw_w, vw_b = _compose(lp["v"]["author"], lp["m_rel"]["writes"])
        kc_w, kc_b = _compose(lp["k"]["paper"], lp["a_rel"]["cites"],
                              scale=lp["p_rel"]["cites"] / _SQRT_D)
        vc_w, vc_b = _compose(lp["v"]["paper"], lp["m_rel"]["cites"])
        kr_w, kr_b = _compose(lp["k"]["paper"], lp["a_rel"]["rev_writes"],
                              scale=lp["p_rel"]["rev_writes"] / _SQRT_D)
        vr_w, vr_b = _compose(lp["v"]["paper"], lp["m_rel"]["rev_writes"])

        wcat_a = jnp.concatenate([lp["q"]["author"]["W"], kw_w, vw_w], axis=1)
        bcat_a = jnp.concatenate([lp["q"]["author"]["b"], kw_b, vw_b])
        wcat_p = jnp.concatenate(
            [lp["q"]["paper"]["W"], kc_w, vc_w, kr_w, vr_w], axis=1)
        bcat_p = jnp.concatenate(
            [lp["q"]["paper"]["b"], kc_b, vc_b, kr_b, vr_b])

        proj_a = _matmul(h_a, wcat_a, bcat_a)
        proj_p = _matmul(h_p, wcat_p, bcat_p)

        q_a = proj_a[:, :128]
        k_w = proj_a[:, 128:256]
        v_w = proj_a[:, 256:384]
        q_p = proj_p[:, :128]
        k_c = proj_p[:, 128:256]
        v_c = proj_p[:, 256:384]
        k_r = proj_p[:, 384:512]
        v_r = proj_p[:, 512:640]

        num_w, den_w = _edge_pass(q_p, k_w, v_w, src_w, dst_w, n_p)
        num_c, den_c = _edge_pass(q_p, k_c, v_c, src_c, dst_c, n_p)
        num_r, den_r = _edge_pass(q_a, k_r, v_r, src_r, dst_r, n_a)

        h_a = _out_stage([(num_r, den_r)], h_a, lp["a"]["author"]["W"],
                         lp["a"]["author"]["b"],
                         jax.nn.sigmoid(lp["skip"]["author"]))
        h_p = _out_stage([(num_w, den_w), (num_c, den_c)], h_p,
                         lp["a"]["paper"]["W"], lp["a"]["paper"]["b"],
                         jax.nn.sigmoid(lp["skip"]["paper"]))

    return h_a, h_p
